# baseline (speedup 1.0000x reference)
import functools

import jax
import jax.numpy as jnp
from jax import lax
from jax.experimental import pallas as pl
from jax.experimental.pallas import tpu as pltpu
from jax.experimental.pallas import tpu_sc as plsc

C = 64
NX, NY = 432, 496
NPOS = NX * NY
NXP = 512
PPOS = NY * NXP
CW = C * PPOS
TOT = CW + PPOS
P = 4512
L = 16
NC, NS = 2, 16
NW = NC * NS
W = NPOS // NW
MZ = PPOS // NW
PV = P // L
PID_BITS = 13
KEY_SENT = 1 << 26
CAPW = 256
SCH = 1024
NSCW = CAPW * C + SCH

_MESH = plsc.VectorSubcoreMesh(
    core_axis_name="c", subcore_axis_name="s", num_cores=NC, num_subcores=NS
)


@functools.partial(
    pl.kernel,
    out_type=jax.ShapeDtypeStruct((TOT,), jnp.float32),
    mesh=_MESH,
    scratch_types=[
        pltpu.VMEM((P * 4,), jnp.int32),
        pltpu.VMEM((P + 2 * L,), jnp.int32),
        pltpu.VMEM((P + 2 * L,), jnp.int32),
        pltpu.VMEM((P + 2 * L,), jnp.int32),
        pltpu.VMEM((P + 2 * L,), jnp.int32),
        pltpu.VMEM((W,), jnp.int32),
        pltpu.VMEM((P + 2 * L,), jnp.int32),
        pltpu.VMEM((P + 2 * L,), jnp.int32),
        pltpu.VMEM((CAPW, 2 * C), jnp.float32),
        pltpu.VMEM((NSCW,), jnp.int32),
        pltpu.VMEM((NSCW,), jnp.float32),
        pltpu.VMEM((2 * L,), jnp.int32),
        pltpu.SemaphoreType.DMA,
        pltpu.SemaphoreType.DMA,
    ],
    compiler_params=pltpu.CompilerParams(needs_layout_passes=False),
)
def _scatter_kernel(pf_hbm, coords_hbm, out_hbm,
                    coords_v, keyw, colb, pidb, iwinb, wslab, pposb, wpidb,
                    rowbuf, idxsc, paysc, nxtb, sem_row, sem_sc):
    w = lax.axis_index("s") * NC + lax.axis_index("c")
    lo = w * W
    iota = lax.iota(jnp.int32, L)
    zeros16i = jnp.zeros((L,), jnp.int32)
    zeros16f = jnp.zeros((L,), jnp.float32)

    def zsrc(r, _):
        paysc[pl.ds(r * L, L)] = zeros16f
        return 0
    lax.fori_loop(0, MZ // L, zsrc, 0)
    pltpu.sync_copy(paysc.at[pl.ds(0, MZ)], out_hbm.at[pl.ds(CW + w * MZ, MZ)])

    pltpu.sync_copy(coords_hbm, coords_v)

    def scan_body(i, cnt):
        p0 = i * L
        base4 = (p0 + iota) * 4
        c1 = plsc.load_gather(coords_v, [base4 + 1])
        c2 = plsc.load_gather(coords_v, [base4 + 2])
        c3 = plsc.load_gather(coords_v, [base4 + 3])
        idx = c1 + c2 * NX + c3
        m = (idx >= lo) & (idx < lo + W)
        key = (idx - lo) * (1 << PID_BITS) + (p0 + iota)
        mi = m.astype(jnp.int32)
        pos = cnt + plsc.cumsum(mi) - 1
        pos = jnp.where(m, pos, 0)
        plsc.store_scatter(keyw, [pos], key, mask=m)
        return cnt + jnp.sum(mi)

    nmemb = lax.fori_loop(0, PV, scan_body, jnp.int32(0))
    plsc.store_scatter(keyw, [nmemb + iota],
                       jnp.full((L,), KEY_SENT, jnp.int32))
    ngrp = (nmemb + L - 1) // L
    nxtb[pl.ds(L, L)] = jnp.full((L,), KEY_SENT, jnp.int32)

    def prep(g, _):
        kv = keyw[pl.ds(g * L, L)]
        sk, _sv = plsc.sort_key_val(kv, kv)
        nxtb[pl.ds(0, L)] = sk
        nxt = nxtb[pl.ds(1, L)]
        win = ((sk >> PID_BITS) != (nxt >> PID_BITS)) & (sk < KEY_SENT)
        col = jnp.minimum(sk >> PID_BITS, W - 1)
        pid = sk & ((1 << PID_BITS) - 1)
        at = g * L + iota
        plsc.store_scatter(colb, [at], col)
        plsc.store_scatter(pidb, [at], pid)
        plsc.store_scatter(iwinb, [at], win.astype(jnp.int32))
        plsc.store_scatter(wslab, [col], pid, mask=win)
        return 0

    lax.fori_loop(0, ngrp, prep, 0)

    def wchk(g, cnt):
        at = g * L + iota
        col = plsc.load_gather(colb, [at])
        pid = plsc.load_gather(pidb, [at])
        iwin = plsc.load_gather(iwinb, [at]) != 0
        gwin = iwin & (plsc.load_gather(wslab, [col]) == pid)
        j = col + lo
        y = ((j.astype(jnp.float32) + 0.5) * (1.0 / NX)).astype(jnp.int32)
        ppos = j + y * (NXP - NX)
        gi = gwin.astype(jnp.int32)
        pos = cnt + plsc.cumsum(gi) - 1
        pos = jnp.where(gwin, pos, 0)
        plsc.store_scatter(pposb, [pos], ppos, mask=gwin)
        plsc.store_scatter(wpidb, [pos], pid, mask=gwin)
        return cnt + jnp.sum(gi)

    nwin = lax.fori_loop(0, ngrp, wchk, jnp.int32(0))

    @pl.when(nwin > 0)
    def _():
        q0 = plsc.load_gather(pposb, [zeros16i])
        p0 = plsc.load_gather(wpidb, [zeros16i])
        plsc.store_scatter(pposb, [nwin + iota], q0)
        plsc.store_scatter(wpidb, [nwin + iota], p0)

        nwin_pad = ((nwin + L - 1) // L) * L
        nbatch = (nwin_pad + CAPW - 1) // CAPW

        def batch_body(b, _):
            r0 = b * CAPW
            nb = jnp.minimum(nwin_pad - r0, CAPW)
            nrch = nb // L

            def gfire(ch, _):
                pidv = wpidb[pl.ds(r0 + ch * L, L)]
                pltpu.async_copy(pf_hbm.at[pidv >> 1],
                                 rowbuf.at[pl.ds(ch * L, L), :], sem_row)
                return 0
            lax.fori_loop(0, nrch, gfire, 0)

            def gdrain(ch, _):
                pltpu.make_async_copy(
                    pf_hbm.at[pl.ds(0, L), :],
                    rowbuf.at[pl.ds(ch * L, L), :], sem_row).wait()
                return 0
            lax.fori_loop(0, nrch, gdrain, 0)

            def bld(ch, _):
                rl = ch * L + iota
                qv = pposb[pl.ds(r0 + ch * L, L)]
                pidv = wpidb[pl.ds(r0 + ch * L, L)]
                half = (pidv & 1) * C
                for c in range(C):
                    t0 = c * nb + ch * L
                    idxsc[pl.ds(t0, L)] = qv + c * PPOS
                    pay = plsc.load_gather(rowbuf, [rl, half + c])
                    paysc[pl.ds(t0, L)] = pay
                idxsc[pl.ds(C * nb + ch * L, L)] = qv + CW
                paysc[pl.ds(C * nb + ch * L, L)] = zeros16f + 1.0
                return 0
            lax.fori_loop(0, nrch, bld, 0)

            qm = plsc.load_gather(pposb, [zeros16i + r0]) + CW

            def padm(k, _):
                at = C * nb + nb + k * L
                idxsc[pl.ds(at, L)] = qm
                paysc[pl.ds(at, L)] = zeros16f + 1.0
                return 0
            lax.fori_loop(0, (SCH - nb) // L, padm, 0)

            nch = (C * nb + SCH) // SCH

            def sfire(ch, _):
                pltpu.async_copy(
                    paysc.at[pl.ds(ch * SCH, SCH)],
                    out_hbm.at[idxsc.at[pl.ds(ch * SCH, SCH)]], sem_sc)
                return 0
            lax.fori_loop(0, nch, sfire, 0)

            def sdrain(ch, _):
                pltpu.make_async_copy(
                    paysc.at[pl.ds(ch * SCH, SCH)],
                    out_hbm.at[idxsc.at[pl.ds(ch * SCH, SCH)]],
                    sem_sc).wait()
                return 0
            lax.fori_loop(0, nch, sdrain, 0)
            return 0

        lax.fori_loop(0, nbatch, batch_body, 0)


_RPP = PPOS // 128


def _fin_body(c_ref, m_ref, o_ref):
    cv = c_ref[...]
    m = m_ref[...]
    mm = jnp.concatenate([m, m, m, m], axis=0)
    o = jnp.where(mm != 0.0, cv, 0.0)
    o4 = o.reshape(4, NY, 4, 128)
    o_ref[0, :, :, 0:128] = o4[:, :, 0, :]
    o_ref[0, :, :, 128:256] = o4[:, :, 1, :]
    o_ref[0, :, :, 256:384] = o4[:, :, 2, :]
    o_ref[0, :, :, 384:NX] = o4[:, :, 3, 0:(NX - 384)]


_finalize = pl.pallas_call(
    _fin_body,
    out_shape=jax.ShapeDtypeStruct((1, C, NY, NX), jnp.float32),
    grid=(C // 4,),
    in_specs=[
        pl.BlockSpec((4 * _RPP, 128), lambda i: (i, 0)),
        pl.BlockSpec((_RPP, 128), lambda i: (C, 0)),
    ],
    out_specs=pl.BlockSpec((1, 4, NY, NX), lambda i: (0, i, 0, 0)),
)


def kernel(pillar_features, coords):
    pf_pairs = pillar_features.reshape(P // 2, 2 * C)
    coords_flat = coords.reshape(P * 4).astype(jnp.int32)
    combined = _scatter_kernel(pf_pairs, coords_flat)
    combined2 = combined.reshape(TOT // 128, 128)
    return _finalize(combined2, combined2)


# speedup vs baseline: 1.0023x; 1.0023x over previous
import functools

import jax
import jax.numpy as jnp
from jax import lax
from jax.experimental import pallas as pl
from jax.experimental.pallas import tpu as pltpu
from jax.experimental.pallas import tpu_sc as plsc

C = 64
NX, NY = 432, 496
NPOS = NX * NY
NXP = 512
PPOS = NY * NXP
CW = C * PPOS
TOT = CW + PPOS
P = 4512
L = 16
NC, NS = 2, 16
NW = NC * NS
W = NPOS // NW
MZ = PPOS // NW
PV = P // L
PID_BITS = 13
KEY_SENT = 1 << 26
CAPW = 256
SCH = 1024
NSCW = CAPW * C + SCH

_MESH = plsc.VectorSubcoreMesh(
    core_axis_name="c", subcore_axis_name="s", num_cores=NC, num_subcores=NS
)


@functools.partial(
    pl.kernel,
    out_type=jax.ShapeDtypeStruct((TOT,), jnp.float32),
    mesh=_MESH,
    scratch_types=[
        pltpu.VMEM((P * 4,), jnp.int32),
        pltpu.VMEM((P + 2 * L,), jnp.int32),
        pltpu.VMEM((P + 2 * L,), jnp.int32),
        pltpu.VMEM((P + 2 * L,), jnp.int32),
        pltpu.VMEM((P + 2 * L,), jnp.int32),
        pltpu.VMEM((W,), jnp.int32),
        pltpu.VMEM((P + 2 * L,), jnp.int32),
        pltpu.VMEM((P + 2 * L,), jnp.int32),
        pltpu.VMEM((CAPW, 2 * C), jnp.float32),
        pltpu.VMEM((NSCW,), jnp.int32),
        pltpu.VMEM((NSCW,), jnp.float32),
        pltpu.VMEM((2 * L,), jnp.int32),
        pltpu.SemaphoreType.DMA,
        pltpu.SemaphoreType.DMA,
    ],
    compiler_params=pltpu.CompilerParams(needs_layout_passes=False),
)
def _scatter_kernel(pf_hbm, coords_hbm, out_hbm,
                    coords_v, keyw, colb, pidb, iwinb, wslab, pposb, wpidb,
                    rowbuf, idxsc, paysc, nxtb, sem_row, sem_sc):
    w = lax.axis_index("s") * NC + lax.axis_index("c")
    lo = w * W
    iota = lax.iota(jnp.int32, L)
    zeros16i = jnp.zeros((L,), jnp.int32)
    zeros16f = jnp.zeros((L,), jnp.float32)

    def zsrc(r, _):
        paysc[pl.ds(r * L, L)] = zeros16f
        return 0
    with jax.named_scope("ph_maskzero"):
        lax.fori_loop(0, MZ // L, zsrc, 0)
        pltpu.sync_copy(paysc.at[pl.ds(0, MZ)],
                        out_hbm.at[pl.ds(CW + w * MZ, MZ)])

    with jax.named_scope("ph_coords"):
        pltpu.sync_copy(coords_hbm, coords_v)

    def scan_body(i, cnt):
        p0 = i * L
        base4 = (p0 + iota) * 4
        c1 = plsc.load_gather(coords_v, [base4 + 1])
        c2 = plsc.load_gather(coords_v, [base4 + 2])
        c3 = plsc.load_gather(coords_v, [base4 + 3])
        idx = c1 + c2 * NX + c3
        m = (idx >= lo) & (idx < lo + W)
        key = (idx - lo) * (1 << PID_BITS) + (p0 + iota)
        mi = m.astype(jnp.int32)
        pos = cnt + plsc.cumsum(mi) - 1
        pos = jnp.where(m, pos, 0)
        plsc.store_scatter(keyw, [pos], key, mask=m)
        return cnt + jnp.sum(mi)

    with jax.named_scope("ph_scan"):
        nmemb = lax.fori_loop(0, PV, scan_body, jnp.int32(0))
    plsc.store_scatter(keyw, [nmemb + iota],
                       jnp.full((L,), KEY_SENT, jnp.int32))
    ngrp = (nmemb + L - 1) // L
    nxtb[pl.ds(L, L)] = jnp.full((L,), KEY_SENT, jnp.int32)

    def prep(g, _):
        kv = keyw[pl.ds(g * L, L)]
        sk, _sv = plsc.sort_key_val(kv, kv)
        nxtb[pl.ds(0, L)] = sk
        nxt = nxtb[pl.ds(1, L)]
        win = ((sk >> PID_BITS) != (nxt >> PID_BITS)) & (sk < KEY_SENT)
        col = jnp.minimum(sk >> PID_BITS, W - 1)
        pid = sk & ((1 << PID_BITS) - 1)
        at = g * L + iota
        plsc.store_scatter(colb, [at], col)
        plsc.store_scatter(pidb, [at], pid)
        plsc.store_scatter(iwinb, [at], win.astype(jnp.int32))
        plsc.store_scatter(wslab, [col], pid, mask=win)
        return 0

    with jax.named_scope("ph_prep"):
        lax.fori_loop(0, ngrp, prep, 0)

    def wchk(g, cnt):
        at = g * L + iota
        col = plsc.load_gather(colb, [at])
        pid = plsc.load_gather(pidb, [at])
        iwin = plsc.load_gather(iwinb, [at]) != 0
        gwin = iwin & (plsc.load_gather(wslab, [col]) == pid)
        j = col + lo
        y = ((j.astype(jnp.float32) + 0.5) * (1.0 / NX)).astype(jnp.int32)
        ppos = j + y * (NXP - NX)
        gi = gwin.astype(jnp.int32)
        pos = cnt + plsc.cumsum(gi) - 1
        pos = jnp.where(gwin, pos, 0)
        plsc.store_scatter(pposb, [pos], ppos, mask=gwin)
        plsc.store_scatter(wpidb, [pos], pid, mask=gwin)
        return cnt + jnp.sum(gi)

    with jax.named_scope("ph_wchk"):
        nwin = lax.fori_loop(0, ngrp, wchk, jnp.int32(0))

    @pl.when(nwin > 0)
    def _():
        q0 = plsc.load_gather(pposb, [zeros16i])
        p0 = plsc.load_gather(wpidb, [zeros16i])
        plsc.store_scatter(pposb, [nwin + iota], q0)
        plsc.store_scatter(wpidb, [nwin + iota], p0)

        nwin_pad = ((nwin + L - 1) // L) * L
        nbatch = (nwin_pad + CAPW - 1) // CAPW

        def batch_body(b, _):
            r0 = b * CAPW
            nb = jnp.minimum(nwin_pad - r0, CAPW)
            nrch = nb // L

            def gfire(ch, _):
                pidv = wpidb[pl.ds(r0 + ch * L, L)]
                pltpu.async_copy(pf_hbm.at[pidv >> 1],
                                 rowbuf.at[pl.ds(ch * L, L), :], sem_row)
                return 0
            with jax.named_scope("ph_gfire"):
                lax.fori_loop(0, nrch, gfire, 0)

            def gdrain(ch, _):
                pltpu.make_async_copy(
                    pf_hbm.at[pl.ds(0, L), :],
                    rowbuf.at[pl.ds(ch * L, L), :], sem_row).wait()
                return 0
            with jax.named_scope("ph_gdrain"):
                lax.fori_loop(0, nrch, gdrain, 0)

            def bld(ch, _):
                rl = ch * L + iota
                qv = pposb[pl.ds(r0 + ch * L, L)]
                pidv = wpidb[pl.ds(r0 + ch * L, L)]
                half = (pidv & 1) * C
                for c in range(C):
                    t0 = c * nb + ch * L
                    idxsc[pl.ds(t0, L)] = qv + c * PPOS
                    pay = plsc.load_gather(rowbuf, [rl, half + c])
                    paysc[pl.ds(t0, L)] = pay
                idxsc[pl.ds(C * nb + ch * L, L)] = qv + CW
                paysc[pl.ds(C * nb + ch * L, L)] = zeros16f + 1.0
                return 0
            with jax.named_scope("ph_bld"):
                lax.fori_loop(0, nrch, bld, 0)

            qm = plsc.load_gather(pposb, [zeros16i + r0]) + CW

            def padm(k, _):
                at = C * nb + nb + k * L
                idxsc[pl.ds(at, L)] = qm
                paysc[pl.ds(at, L)] = zeros16f + 1.0
                return 0
            with jax.named_scope("ph_padm"):
                lax.fori_loop(0, (SCH - nb) // L, padm, 0)

            nch = (C * nb + SCH) // SCH

            def sfire(ch, _):
                pltpu.async_copy(
                    paysc.at[pl.ds(ch * SCH, SCH)],
                    out_hbm.at[idxsc.at[pl.ds(ch * SCH, SCH)]], sem_sc)
                return 0
            with jax.named_scope("ph_sfire"):
                lax.fori_loop(0, nch, sfire, 0)

            def sdrain(ch, _):
                pltpu.make_async_copy(
                    paysc.at[pl.ds(ch * SCH, SCH)],
                    out_hbm.at[idxsc.at[pl.ds(ch * SCH, SCH)]],
                    sem_sc).wait()
                return 0
            with jax.named_scope("ph_sdrain"):
                lax.fori_loop(0, nch, sdrain, 0)
            return 0

        lax.fori_loop(0, nbatch, batch_body, 0)


_RPP = PPOS // 128


def _fin_body(c_ref, m_ref, o_ref):
    cv = c_ref[...]
    m = m_ref[...]
    mm = jnp.concatenate([m, m, m, m], axis=0)
    o = jnp.where(mm != 0.0, cv, 0.0)
    o4 = o.reshape(4, NY, 4, 128)
    o_ref[0, :, :, 0:128] = o4[:, :, 0, :]
    o_ref[0, :, :, 128:256] = o4[:, :, 1, :]
    o_ref[0, :, :, 256:384] = o4[:, :, 2, :]
    o_ref[0, :, :, 384:NX] = o4[:, :, 3, 0:(NX - 384)]


_finalize = pl.pallas_call(
    _fin_body,
    out_shape=jax.ShapeDtypeStruct((1, C, NY, NX), jnp.float32),
    grid=(C // 4,),
    in_specs=[
        pl.BlockSpec((4 * _RPP, 128), lambda i: (i, 0)),
        pl.BlockSpec((_RPP, 128), lambda i: (C, 0)),
    ],
    out_specs=pl.BlockSpec((1, 4, NY, NX), lambda i: (0, i, 0, 0)),
)


def kernel(pillar_features, coords):
    pf_pairs = pillar_features.reshape(P // 2, 2 * C)
    coords_flat = coords.reshape(P * 4).astype(jnp.int32)
    combined = _scatter_kernel(pf_pairs, coords_flat)
    combined2 = combined.reshape(TOT // 128, 128)
    return _finalize(combined2, combined2)


# speedup vs baseline: 3.4557x; 3.4476x over previous
import functools

import jax
import jax.numpy as jnp
from jax import lax
from jax.experimental import pallas as pl
from jax.experimental.pallas import tpu as pltpu
from jax.experimental.pallas import tpu_sc as plsc

C = 64
NX, NY = 432, 496
NPOS = NX * NY
NXP = 512
PPOS = NY * NXP
P = 4512
L = 16
NC, NS = 2, 16
NW = NC * NS
W = NPOS // NW
MZ = PPOS // NW
PV = P // L
PID_BITS = 13
KEY_SENT = 1 << 26
CAPW = 256

_MESH = plsc.VectorSubcoreMesh(
    core_axis_name="c", subcore_axis_name="s", num_cores=NC, num_subcores=NS
)


@functools.partial(
    pl.kernel,
    out_type=(
        jax.ShapeDtypeStruct((PPOS, 2 * C), jnp.float32),
        jax.ShapeDtypeStruct((PPOS,), jnp.float32),
    ),
    mesh=_MESH,
    scratch_types=[
        pltpu.VMEM((P * 4,), jnp.int32),
        pltpu.VMEM((P + 2 * L,), jnp.int32),
        pltpu.VMEM((P + 2 * L,), jnp.int32),
        pltpu.VMEM((P + 2 * L,), jnp.int32),
        pltpu.VMEM((P + 2 * L,), jnp.int32),
        pltpu.VMEM((W,), jnp.int32),
        pltpu.VMEM((P + 2 * L,), jnp.int32),
        pltpu.VMEM((P + 2 * L,), jnp.int32),
        pltpu.VMEM((CAPW, 2 * C), jnp.float32),
        pltpu.VMEM((CAPW, 2 * C), jnp.float32),
        pltpu.VMEM((MZ,), jnp.float32),
        pltpu.VMEM((2 * L,), jnp.int32),
        pltpu.SemaphoreType.DMA,
        pltpu.SemaphoreType.DMA,
    ],
    compiler_params=pltpu.CompilerParams(needs_layout_passes=False),
)
def _scatter_kernel(pf_hbm, coords_hbm, wt_hbm, mk_hbm,
                    coords_v, keyw, colb, pidb, iwinb, wslab, pposb, wpidb,
                    rowbuf, paybuf, zob, nxtb, sem_row, sem_sc):
    w = lax.axis_index("s") * NC + lax.axis_index("c")
    lo = w * W
    iota = lax.iota(jnp.int32, L)
    zeros16i = jnp.zeros((L,), jnp.int32)
    zeros16f = jnp.zeros((L,), jnp.float32)

    def zsrc(r, _):
        zob[pl.ds(r * L, L)] = zeros16f
        return 0
    lax.fori_loop(0, MZ // L, zsrc, 0)
    pltpu.sync_copy(zob, mk_hbm.at[pl.ds(w * MZ, MZ)])
    for r in range(8):
        zob[pl.ds(r * L, L)] = zeros16f + 1.0

    pltpu.sync_copy(coords_hbm, coords_v)

    def scan_body(i, cnt):
        p0 = i * L
        base4 = (p0 + iota) * 4
        c1 = plsc.load_gather(coords_v, [base4 + 1])
        c2 = plsc.load_gather(coords_v, [base4 + 2])
        c3 = plsc.load_gather(coords_v, [base4 + 3])
        idx = c1 + c2 * NX + c3
        m = (idx >= lo) & (idx < lo + W)
        key = (idx - lo) * (1 << PID_BITS) + (p0 + iota)
        mi = m.astype(jnp.int32)
        pos = cnt + plsc.cumsum(mi) - 1
        pos = jnp.where(m, pos, 0)
        plsc.store_scatter(keyw, [pos], key, mask=m)
        return cnt + jnp.sum(mi)

    nmemb = lax.fori_loop(0, PV, scan_body, jnp.int32(0))
    plsc.store_scatter(keyw, [nmemb + iota],
                       jnp.full((L,), KEY_SENT, jnp.int32))
    ngrp = (nmemb + L - 1) // L
    nxtb[pl.ds(L, L)] = jnp.full((L,), KEY_SENT, jnp.int32)

    def prep(g, _):
        kv = keyw[pl.ds(g * L, L)]
        sk, _sv = plsc.sort_key_val(kv, kv)
        nxtb[pl.ds(0, L)] = sk
        nxt = nxtb[pl.ds(1, L)]
        win = ((sk >> PID_BITS) != (nxt >> PID_BITS)) & (sk < KEY_SENT)
        col = jnp.minimum(sk >> PID_BITS, W - 1)
        pid = sk & ((1 << PID_BITS) - 1)
        at = g * L + iota
        plsc.store_scatter(colb, [at], col)
        plsc.store_scatter(pidb, [at], pid)
        plsc.store_scatter(iwinb, [at], win.astype(jnp.int32))
        plsc.store_scatter(wslab, [col], pid, mask=win)
        return 0

    lax.fori_loop(0, ngrp, prep, 0)

    def wchk(g, cnt):
        at = g * L + iota
        col = plsc.load_gather(colb, [at])
        pid = plsc.load_gather(pidb, [at])
        iwin = plsc.load_gather(iwinb, [at]) != 0
        gwin = iwin & (plsc.load_gather(wslab, [col]) == pid)
        j = col + lo
        y = ((j.astype(jnp.float32) + 0.5) * (1.0 / NX)).astype(jnp.int32)
        ppos = j + y * (NXP - NX)
        gi = gwin.astype(jnp.int32)
        pos = cnt + plsc.cumsum(gi) - 1
        pos = jnp.where(gwin, pos, 0)
        plsc.store_scatter(pposb, [pos], ppos, mask=gwin)
        plsc.store_scatter(wpidb, [pos], pid, mask=gwin)
        return cnt + jnp.sum(gi)

    nwin = lax.fori_loop(0, ngrp, wchk, jnp.int32(0))

    @pl.when(nwin > 0)
    def _():
        q0 = plsc.load_gather(pposb, [zeros16i])
        p0 = plsc.load_gather(wpidb, [zeros16i])
        for r in range(8):
            plsc.store_scatter(pposb, [nwin + r * L + iota], q0)
            plsc.store_scatter(wpidb, [nwin + r * L + iota], p0)

        nwin_pad = ((nwin + L - 1) // L) * L
        nbatch = (nwin_pad + CAPW - 1) // CAPW

        def batch_body(b, _):
            r0 = b * CAPW
            nb = jnp.minimum(nwin_pad - r0, CAPW)
            nqch = (nb + 127) // 128
            nrch = nqch * 8

            def gfire(ch, _):
                pidv = wpidb[pl.ds(r0 + ch * L, L)]
                pltpu.async_copy(pf_hbm.at[pidv >> 1],
                                 rowbuf.at[pl.ds(ch * L, L), :], sem_row)
                return 0
            lax.fori_loop(0, nrch, gfire, 0)

            def gdrain(ch, _):
                pltpu.make_async_copy(
                    pf_hbm.at[pl.ds(0, L), :],
                    rowbuf.at[pl.ds(ch * L, L), :], sem_row).wait()
                return 0
            lax.fori_loop(0, nrch, gdrain, 0)

            def bld(ch, _):
                rl = ch * L + iota
                pidv = wpidb[pl.ds(r0 + ch * L, L)]
                half = (pidv & 1) * C
                for c in range(C):
                    pay = plsc.load_gather(rowbuf, [rl, half + c])
                    plsc.store_scatter(paybuf, [rl, jnp.full((L,), c,
                                                             jnp.int32)], pay)
                return 0
            lax.fori_loop(0, nrch, bld, 0)

            def sfire(k, _):
                rows = pl.ds(k * 128, 128)
                pltpu.async_copy(paybuf.at[rows, :],
                                 wt_hbm.at[pposb.at[pl.ds(r0 + k * 128, 128)]],
                                 sem_sc)
                pltpu.async_copy(zob.at[pl.ds(0, 128)],
                                 mk_hbm.at[pposb.at[pl.ds(r0 + k * 128, 128)]],
                                 sem_sc)
                return 0
            lax.fori_loop(0, nqch, sfire, 0)

            def sdrain(k, _):
                rows = pl.ds(k * 128, 128)
                pltpu.make_async_copy(
                    paybuf.at[rows, :],
                    wt_hbm.at[pposb.at[pl.ds(r0 + k * 128, 128)]],
                    sem_sc).wait()
                pltpu.make_async_copy(
                    zob.at[pl.ds(0, 128)],
                    mk_hbm.at[pposb.at[pl.ds(r0 + k * 128, 128)]],
                    sem_sc).wait()
                return 0
            lax.fori_loop(0, nqch, sdrain, 0)
            return 0

        lax.fori_loop(0, nbatch, batch_body, 0)


def _fin_body(w_ref, m_ref, o_ref):
    wt = w_ref[...]
    m2 = m_ref[...]
    a = wt[:, 0:C]
    a3 = a.T.reshape(C, 16, NXP)
    m3 = m2.reshape(16, 4, 128)
    m512 = jnp.concatenate(
        [m3[:, 0, :], m3[:, 1, :], m3[:, 2, :], m3[:, 3, :]], axis=-1)
    o = jnp.where(m512[None] != 0.0, a3, 0.0)
    o_ref[...] = o[:, :, :NX][None]


_finalize = pl.pallas_call(
    _fin_body,
    out_shape=jax.ShapeDtypeStruct((1, C, NY, NX), jnp.float32),
    grid=(NY // 16,),
    in_specs=[
        pl.BlockSpec((16 * NXP, 2 * C), lambda i: (i, 0)),
        pl.BlockSpec((16 * NXP // 128, 128), lambda i: (i, 0)),
    ],
    out_specs=pl.BlockSpec((1, C, 16, NX), lambda i: (0, 0, i, 0)),
)


def kernel(pillar_features, coords):
    pf_pairs = pillar_features.reshape(P // 2, 2 * C)
    coords_flat = coords.reshape(P * 4).astype(jnp.int32)
    wt, mk = _scatter_kernel(pf_pairs, coords_flat)
    mk2 = mk.reshape(PPOS // 128, 128)
    return _finalize(wt, mk2)


# speedup vs baseline: 5.3281x; 1.5418x over previous
import functools

import jax
import jax.numpy as jnp
from jax import lax
from jax.experimental import pallas as pl
from jax.experimental.pallas import tpu as pltpu
from jax.experimental.pallas import tpu_sc as plsc

C = 64
NX, NY = 432, 496
NPOS = NX * NY
NXP = 512
PPOS = NX * NXP
P = 4512
L = 16
NC, NS = 2, 16
NW = NC * NS
W = NPOS // NW
MZ = PPOS // NW
PV = P // L
PID_BITS = 13
KEY_SENT = 1 << 26
CAPW = 256

_MESH = plsc.VectorSubcoreMesh(
    core_axis_name="c", subcore_axis_name="s", num_cores=NC, num_subcores=NS
)


@functools.partial(
    pl.kernel,
    out_type=(
        jax.ShapeDtypeStruct((PPOS, 2 * C), jnp.float32),
        jax.ShapeDtypeStruct((PPOS,), jnp.float32),
    ),
    mesh=_MESH,
    scratch_types=[
        pltpu.VMEM((P * 4,), jnp.int32),
        pltpu.VMEM((P + 2 * L,), jnp.int32),
        pltpu.VMEM((P + 2 * L,), jnp.int32),
        pltpu.VMEM((P + 2 * L,), jnp.int32),
        pltpu.VMEM((P + 2 * L,), jnp.int32),
        pltpu.VMEM((W,), jnp.int32),
        pltpu.VMEM((P + 2 * L,), jnp.int32),
        pltpu.VMEM((P + 2 * L,), jnp.int32),
        pltpu.VMEM((CAPW, 2 * C), jnp.float32),
        pltpu.VMEM((CAPW, 2 * C), jnp.float32),
        pltpu.VMEM((MZ,), jnp.float32),
        pltpu.VMEM((2 * L,), jnp.int32),
        pltpu.SemaphoreType.DMA,
        pltpu.SemaphoreType.DMA,
    ],
    compiler_params=pltpu.CompilerParams(needs_layout_passes=False),
)
def _scatter_kernel(pf_hbm, coords_hbm, wt_hbm, mk_hbm,
                    coords_v, keyw, colb, pidb, iwinb, wslab, pposb, wpidb,
                    rowbuf, paybuf, zob, nxtb, sem_row, sem_sc):
    w = lax.axis_index("s") * NC + lax.axis_index("c")
    lo = w * W
    iota = lax.iota(jnp.int32, L)
    zeros16i = jnp.zeros((L,), jnp.int32)
    zeros16f = jnp.zeros((L,), jnp.float32)

    def zsrc(r, _):
        zob[pl.ds(r * L, L)] = zeros16f
        return 0
    lax.fori_loop(0, MZ // L, zsrc, 0)
    pltpu.sync_copy(zob, mk_hbm.at[pl.ds(w * MZ, MZ)])
    for r in range(8):
        zob[pl.ds(r * L, L)] = zeros16f + 1.0

    pltpu.sync_copy(coords_hbm, coords_v)

    def scan_body(i, cnt):
        p0 = i * L
        base4 = (p0 + iota) * 4
        c1 = plsc.load_gather(coords_v, [base4 + 1])
        c2 = plsc.load_gather(coords_v, [base4 + 2])
        c3 = plsc.load_gather(coords_v, [base4 + 3])
        idx = c1 + c2 * NX + c3
        m = (idx >= lo) & (idx < lo + W)
        key = (idx - lo) * (1 << PID_BITS) + (p0 + iota)
        mi = m.astype(jnp.int32)
        pos = cnt + plsc.cumsum(mi) - 1
        pos = jnp.where(m, pos, 0)
        plsc.store_scatter(keyw, [pos], key, mask=m)
        return cnt + jnp.sum(mi)

    nmemb = lax.fori_loop(0, PV, scan_body, jnp.int32(0))
    plsc.store_scatter(keyw, [nmemb + iota],
                       jnp.full((L,), KEY_SENT, jnp.int32))
    ngrp = (nmemb + L - 1) // L
    nxtb[pl.ds(L, L)] = jnp.full((L,), KEY_SENT, jnp.int32)

    def prep(g, _):
        kv = keyw[pl.ds(g * L, L)]
        sk, _sv = plsc.sort_key_val(kv, kv)
        nxtb[pl.ds(0, L)] = sk
        nxt = nxtb[pl.ds(1, L)]
        win = ((sk >> PID_BITS) != (nxt >> PID_BITS)) & (sk < KEY_SENT)
        col = jnp.minimum(sk >> PID_BITS, W - 1)
        pid = sk & ((1 << PID_BITS) - 1)
        at = g * L + iota
        plsc.store_scatter(colb, [at], col)
        plsc.store_scatter(pidb, [at], pid)
        plsc.store_scatter(iwinb, [at], win.astype(jnp.int32))
        plsc.store_scatter(wslab, [col], pid, mask=win)
        return 0

    lax.fori_loop(0, ngrp, prep, 0)

    def wchk(g, cnt):
        at = g * L + iota
        col = plsc.load_gather(colb, [at])
        pid = plsc.load_gather(pidb, [at])
        iwin = plsc.load_gather(iwinb, [at]) != 0
        gwin = iwin & (plsc.load_gather(wslab, [col]) == pid)
        j = col + lo
        y = ((j.astype(jnp.float32) + 0.5) * (1.0 / NX)).astype(jnp.int32)
        x = j - y * NX
        ppos = x * NXP + y
        gi = gwin.astype(jnp.int32)
        pos = cnt + plsc.cumsum(gi) - 1
        pos = jnp.where(gwin, pos, 0)
        plsc.store_scatter(pposb, [pos], ppos, mask=gwin)
        plsc.store_scatter(wpidb, [pos], pid, mask=gwin)
        return cnt + jnp.sum(gi)

    nwin = lax.fori_loop(0, ngrp, wchk, jnp.int32(0))

    @pl.when(nwin > 0)
    def _():
        q0 = plsc.load_gather(pposb, [zeros16i])
        p0 = plsc.load_gather(wpidb, [zeros16i])
        for r in range(8):
            plsc.store_scatter(pposb, [nwin + r * L + iota], q0)
            plsc.store_scatter(wpidb, [nwin + r * L + iota], p0)

        nwin_pad = ((nwin + L - 1) // L) * L
        nbatch = (nwin_pad + CAPW - 1) // CAPW

        def batch_body(b, _):
            r0 = b * CAPW
            nb = jnp.minimum(nwin_pad - r0, CAPW)
            nqch = (nb + 127) // 128
            nrch = nqch * 8

            def gfire(ch, _):
                pidv = wpidb[pl.ds(r0 + ch * L, L)]
                pltpu.async_copy(pf_hbm.at[pidv >> 1],
                                 rowbuf.at[pl.ds(ch * L, L), :], sem_row)
                return 0
            lax.fori_loop(0, nrch, gfire, 0)

            def gdrain(ch, _):
                pltpu.make_async_copy(
                    pf_hbm.at[pl.ds(0, L), :],
                    rowbuf.at[pl.ds(ch * L, L), :], sem_row).wait()
                return 0
            lax.fori_loop(0, nrch, gdrain, 0)

            def bld(ch, _):
                rl = ch * L + iota
                pidv = wpidb[pl.ds(r0 + ch * L, L)]
                half = (pidv & 1) * C
                for c in range(C):
                    pay = plsc.load_gather(rowbuf, [rl, half + c])
                    plsc.store_scatter(paybuf, [rl, jnp.full((L,), c,
                                                             jnp.int32)], pay)
                return 0
            lax.fori_loop(0, nrch, bld, 0)

            def sfire(k, _):
                rows = pl.ds(k * 128, 128)
                pltpu.async_copy(paybuf.at[rows, :],
                                 wt_hbm.at[pposb.at[pl.ds(r0 + k * 128, 128)]],
                                 sem_sc)
                pltpu.async_copy(zob.at[pl.ds(0, 128)],
                                 mk_hbm.at[pposb.at[pl.ds(r0 + k * 128, 128)]],
                                 sem_sc)
                return 0
            lax.fori_loop(0, nqch, sfire, 0)

            def sdrain(k, _):
                rows = pl.ds(k * 128, 128)
                pltpu.make_async_copy(
                    paybuf.at[rows, :],
                    wt_hbm.at[pposb.at[pl.ds(r0 + k * 128, 128)]],
                    sem_sc).wait()
                pltpu.make_async_copy(
                    zob.at[pl.ds(0, 128)],
                    mk_hbm.at[pposb.at[pl.ds(r0 + k * 128, 128)]],
                    sem_sc).wait()
                return 0
            lax.fori_loop(0, nqch, sdrain, 0)
            return 0

        lax.fori_loop(0, nbatch, batch_body, 0)


def _fin_body(w_ref, m_ref, o_ref):
    wt = w_ref[...]
    m2 = m_ref[...]
    a = wt[:, 0:C]
    a3 = a.T.reshape(C, 16, NXP)
    m3 = m2.reshape(16, 4, 128)
    m512 = jnp.concatenate(
        [m3[:, 0, :], m3[:, 1, :], m3[:, 2, :], m3[:, 3, :]], axis=-1)
    o = jnp.where(m512[None] != 0.0, a3, 0.0)
    o_ref[...] = o[:, :, :NY][None]


_finalize = pl.pallas_call(
    _fin_body,
    out_shape=jax.ShapeDtypeStruct((1, C, NX, NY), jnp.float32),
    grid=(NX // 16,),
    in_specs=[
        pl.BlockSpec((16 * NXP, 2 * C), lambda i: (i, 0)),
        pl.BlockSpec((16 * NXP // 128, 128), lambda i: (i, 0)),
    ],
    out_specs=pl.BlockSpec((1, C, 16, NY), lambda i: (0, 0, i, 0)),
)


def kernel(pillar_features, coords):
    pf_pairs = pillar_features.reshape(P // 2, 2 * C)
    coords_flat = coords.reshape(P * 4).astype(jnp.int32)
    wt, mk = _scatter_kernel(pf_pairs, coords_flat)
    mk2 = mk.reshape(PPOS // 128, 128)
    return jnp.transpose(_finalize(wt, mk2), (0, 1, 3, 2))
